# baseline (speedup 1.0000x reference)
import functools

import jax
import jax.numpy as jnp
from jax import lax
from jax.experimental import pallas as pl
from jax.experimental.pallas import tpu as pltpu
from jax.experimental.pallas import tpu_sc as plsc

IN_F = 2048
OUT_F = 4096
NCB = 16
NCODES = 256
DSUB = IN_F // NCB
NCL = 32
EPS = 1e-5

NTOK = 2048
NW = 32
TB = 16
TPW = NTOK // NW
NBLK = TPW // TB
FQ = 1024
NQ = OUT_F // FQ
GPQ = FQ // 16


def _pqa_body(x_ref, cb_ref, ln_ref, cent_ref, clcol_ref, bias_ref,
              pt_ref, mask_ref, mbias_ref, dots_ref):
    c = pl.program_id(0)
    x = x_ref[...]
    cb = cb_ref[0]
    p = jax.lax.dot_general(x, cb.astype(jnp.bfloat16), (((1,), (1,)), ((), ())),
                            preferred_element_type=jnp.float32)
    pt_ref[...] = p

    cents = cent_ref[0]
    iota = jax.lax.broadcasted_iota(jnp.int32, (NCL, NCODES), 1)
    oh = (cents[0][:, None] == iota).astype(jnp.bfloat16)
    d = jax.lax.dot_general(p.astype(jnp.bfloat16), oh, (((1,), (1,)), ((), ())),
                            preferred_element_type=jnp.float32)

    @pl.when(c == 0)
    def _():
        dots_ref[...] = jnp.zeros_like(dots_ref)
    dots_ref[...] += d

    @pl.when(c == pl.num_programs(0) - 1)
    def _fin():
        dt = dots_ref[...]
        mean = jnp.mean(dt, axis=1, keepdims=True)
        var = jnp.mean((dt - mean) ** 2, axis=1, keepdims=True)
        s = (dt - mean) * jax.lax.rsqrt(var + EPS) * ln_ref[...]
        rowmax = jnp.max(s, axis=1, keepdims=True)
        iota1 = jax.lax.broadcasted_iota(jnp.int32, (NTOK, NCL), 1)
        am = jnp.min(jnp.where(s >= rowmax, iota1, NCL), axis=1, keepdims=True)
        hit = (am == iota1).astype(jnp.float32)
        sel = jnp.max(hit, axis=0, keepdims=True)
        clc = clcol_ref[...]
        eq = (clc == jax.lax.broadcasted_iota(jnp.int32, (NCL, OUT_F), 0)
              ).astype(jnp.float32)
        m = jnp.max(eq * sel.reshape(NCL, 1), axis=0, keepdims=True)
        mask_ref[...] = m
        mbias_ref[...] = m * bias_ref[...]


def _sc_body(pt_hbm, addr_hbm, mb_hbm, mk_hbm, out_hbm,
             p_v, addr_v, mb_v, mk_v, o_v):
    wid = lax.axis_index("s") * 2 + lax.axis_index("c")

    def blk(b, _):
        t0 = wid * TPW + b * TB
        pltpu.sync_copy(pt_hbm.at[pl.ds(t0, TB), :], p_v)

        def qloop(q, _):
            pltpu.sync_copy(addr_hbm.at[q], addr_v)
            pltpu.sync_copy(mb_hbm.at[q], mb_v)
            pltpu.sync_copy(mk_hbm.at[q], mk_v)

            def grp(g, _):
                a = [addr_v[cc, pl.ds(g * 16, 16)] for cc in range(NCB)]
                mb = mb_v[pl.ds(g * 16, 16)]
                mk = mk_v[pl.ds(g * 16, 16)]
                for t in range(TB):
                    row = jnp.full((16,), t, jnp.int32)
                    acc = plsc.load_gather(p_v, [row, a[0]])
                    for cc in range(1, NCB):
                        acc = acc + plsc.load_gather(p_v, [row, a[cc]])
                    o_v[t, pl.ds(g * 16, 16)] = acc * mk + mb
                return 0

            lax.fori_loop(0, GPQ, grp, 0)
            pltpu.sync_copy(o_v, out_hbm.at[pl.ds(t0, TB), pl.ds(q * FQ, FQ)])
            return 0

        lax.fori_loop(0, NQ, qloop, 0)
        return 0

    lax.fori_loop(0, NBLK, blk, 0)


def kernel(x, codebooks, bias, ln_weight, centroids, codes, indices):
    shape = x.shape
    xin = x.reshape(-1, shape[-1])
    n = xin.shape[0]
    ncl, csz = indices.shape
    xbf = xin.astype(jnp.bfloat16)
    cluster_of_col = jnp.zeros((OUT_F,), jnp.int32).at[indices.reshape(-1)].set(
        jnp.repeat(jnp.arange(ncl, dtype=jnp.int32), csz))
    clc2d = cluster_of_col.reshape(1, OUT_F)
    ln2d = ln_weight.reshape(1, NCL)
    bias2d = bias.reshape(1, OUT_F)
    addrs = (codes.astype(jnp.int32)
             + jnp.arange(NCB, dtype=jnp.int32)[:, None] * NCODES)
    addrs_q = addrs.reshape(NCB, NQ, FQ).transpose(1, 0, 2)

    pt, mask, mbias = pl.pallas_call(
        _pqa_body,
        grid=(NCB,),
        in_specs=[
            pl.BlockSpec((NTOK, DSUB), lambda c: (0, c)),
            pl.BlockSpec((1, NCODES, DSUB), lambda c: (c, 0, 0)),
            pl.BlockSpec((1, NCL), lambda c: (0, 0)),
            pl.BlockSpec((1, 1, NCL), lambda c: (c, 0, 0)),
            pl.BlockSpec((1, OUT_F), lambda c: (0, 0)),
            pl.BlockSpec((1, OUT_F), lambda c: (0, 0)),
        ],
        out_specs=[
            pl.BlockSpec((NTOK, NCODES), lambda c: (0, c)),
            pl.BlockSpec((1, OUT_F), lambda c: (0, 0)),
            pl.BlockSpec((1, OUT_F), lambda c: (0, 0)),
        ],
        out_shape=[
            jax.ShapeDtypeStruct((NTOK, NCB * NCODES), jnp.float32),
            jax.ShapeDtypeStruct((1, OUT_F), jnp.float32),
            jax.ShapeDtypeStruct((1, OUT_F), jnp.float32),
        ],
        scratch_shapes=[pltpu.VMEM((NTOK, NCL), jnp.float32)],
    )(xbf, codebooks, ln2d, centroids.reshape(NCB, 1, NCL), clc2d, bias2d)

    mesh = plsc.VectorSubcoreMesh(core_axis_name="c", subcore_axis_name="s")
    sc = functools.partial(
        pl.kernel,
        mesh=mesh,
        compiler_params=pltpu.CompilerParams(use_tc_tiling_on_sc=False, needs_layout_passes=False),
        out_type=jax.ShapeDtypeStruct((NTOK, OUT_F), jnp.float32),
        scratch_types=[
            pltpu.VMEM((TB, NCB * NCODES), jnp.float32),
            pltpu.VMEM((NCB, FQ), jnp.int32),
            pltpu.VMEM((FQ,), jnp.float32),
            pltpu.VMEM((FQ,), jnp.float32),
            pltpu.VMEM((TB, FQ), jnp.float32),
        ],
    )(_sc_body)
    y = sc(pt, addrs_q, mbias.reshape(NQ, FQ), mask.reshape(NQ, FQ))

    return y.reshape(*shape[:-1], OUT_F)


# speedup vs baseline: 1.3752x; 1.3752x over previous
import functools

import jax
import jax.numpy as jnp
from jax import lax
from jax.experimental import pallas as pl
from jax.experimental.pallas import tpu as pltpu
from jax.experimental.pallas import tpu_sc as plsc

IN_F = 2048
OUT_F = 4096
NCB = 16
NCODES = 256
DSUB = IN_F // NCB
NCL = 32
EPS = 1e-5

NTOK = 2048
NW = 32
TB = 16
TPW = NTOK // NW
NBLK = TPW // TB
FQ = 1024
NQ = OUT_F // FQ
GPQ = FQ // 16


def _pqa_body(x_ref, cb_ref, ln_ref, cent_ref, clcol_ref, bias_ref,
              pt_ref, mask_ref, mbias_ref, dots_ref):
    c = pl.program_id(0)
    x = x_ref[...]
    cb = cb_ref[0]
    p = jax.lax.dot_general(x, cb.astype(jnp.bfloat16), (((1,), (1,)), ((), ())),
                            preferred_element_type=jnp.float32)
    pt_ref[...] = p

    cents = cent_ref[0]
    iota = jax.lax.broadcasted_iota(jnp.int32, (NCL, NCODES), 1)
    oh = (cents[0][:, None] == iota).astype(jnp.bfloat16)
    d = jax.lax.dot_general(p.astype(jnp.bfloat16), oh, (((1,), (1,)), ((), ())),
                            preferred_element_type=jnp.float32)

    @pl.when(c == 0)
    def _():
        dots_ref[...] = jnp.zeros_like(dots_ref)
    dots_ref[...] += d

    @pl.when(c == pl.num_programs(0) - 1)
    def _fin():
        dt = dots_ref[...]
        mean = jnp.mean(dt, axis=1, keepdims=True)
        var = jnp.mean((dt - mean) ** 2, axis=1, keepdims=True)
        s = (dt - mean) * jax.lax.rsqrt(var + EPS) * ln_ref[...]
        rowmax = jnp.max(s, axis=1, keepdims=True)
        iota1 = jax.lax.broadcasted_iota(jnp.int32, (NTOK, NCL), 1)
        am = jnp.min(jnp.where(s >= rowmax, iota1, NCL), axis=1, keepdims=True)
        hit = (am == iota1).astype(jnp.float32)
        sel = jnp.max(hit, axis=0, keepdims=True)
        clc = clcol_ref[...]
        eq = (clc == jax.lax.broadcasted_iota(jnp.int32, (NCL, OUT_F), 0)
              ).astype(jnp.float32)
        m = jnp.max(eq * sel.reshape(NCL, 1), axis=0, keepdims=True)
        mask_ref[...] = m
        mbias_ref[...] = m * bias_ref[...]


def _sc_body(pt_hbm, addr_hbm, mb_hbm, mk_hbm, out_hbm,
             p_v, addr_v, mb_v, mk_v, o_v):
    wid = lax.axis_index("s") * 2 + lax.axis_index("c")

    def blk(b, _):
        t0 = wid * TPW + b * TB
        pltpu.sync_copy(pt_hbm.at[pl.ds(t0, TB), :], p_v)

        def qloop(q, _):
            pltpu.sync_copy(addr_hbm.at[q], addr_v)
            pltpu.sync_copy(mb_hbm.at[q], mb_v)
            pltpu.sync_copy(mk_hbm.at[q], mk_v)

            @plsc.parallel_loop(0, GPQ, unroll=2)
            def grp(g):
                a = [addr_v[cc, pl.ds(g * 16, 16)] for cc in range(NCB)]
                mb = mb_v[pl.ds(g * 16, 16)]
                mk = mk_v[pl.ds(g * 16, 16)]
                for t in range(TB):
                    row = jnp.full((16,), t, jnp.int32)
                    vs = [plsc.load_gather(p_v, [row, a[cc]]) for cc in range(NCB)]
                    while len(vs) > 1:
                        nxt = [vs[i] + vs[i + 1] for i in range(0, len(vs) - 1, 2)]
                        if len(vs) % 2:
                            nxt.append(vs[-1])
                        vs = nxt
                    o_v[t, pl.ds(g * 16, 16)] = vs[0] * mk + mb
            pltpu.sync_copy(o_v, out_hbm.at[pl.ds(t0, TB), pl.ds(q * FQ, FQ)])
            return 0

        lax.fori_loop(0, NQ, qloop, 0)
        return 0

    lax.fori_loop(0, NBLK, blk, 0)


def kernel(x, codebooks, bias, ln_weight, centroids, codes, indices):
    shape = x.shape
    xin = x.reshape(-1, shape[-1])
    n = xin.shape[0]
    ncl, csz = indices.shape
    xbf = xin.astype(jnp.bfloat16)
    cluster_of_col = jnp.zeros((OUT_F,), jnp.int32).at[indices.reshape(-1)].set(
        jnp.repeat(jnp.arange(ncl, dtype=jnp.int32), csz))
    clc2d = cluster_of_col.reshape(1, OUT_F)
    ln2d = ln_weight.reshape(1, NCL)
    bias2d = bias.reshape(1, OUT_F)
    addrs = (codes.astype(jnp.int32)
             + jnp.arange(NCB, dtype=jnp.int32)[:, None] * NCODES)
    addrs_q = addrs.reshape(NCB, NQ, FQ).transpose(1, 0, 2)

    pt, mask, mbias = pl.pallas_call(
        _pqa_body,
        grid=(NCB,),
        in_specs=[
            pl.BlockSpec((NTOK, DSUB), lambda c: (0, c)),
            pl.BlockSpec((1, NCODES, DSUB), lambda c: (c, 0, 0)),
            pl.BlockSpec((1, NCL), lambda c: (0, 0)),
            pl.BlockSpec((1, 1, NCL), lambda c: (c, 0, 0)),
            pl.BlockSpec((1, OUT_F), lambda c: (0, 0)),
            pl.BlockSpec((1, OUT_F), lambda c: (0, 0)),
        ],
        out_specs=[
            pl.BlockSpec((NTOK, NCODES), lambda c: (0, c)),
            pl.BlockSpec((1, OUT_F), lambda c: (0, 0)),
            pl.BlockSpec((1, OUT_F), lambda c: (0, 0)),
        ],
        out_shape=[
            jax.ShapeDtypeStruct((NTOK, NCB * NCODES), jnp.float32),
            jax.ShapeDtypeStruct((1, OUT_F), jnp.float32),
            jax.ShapeDtypeStruct((1, OUT_F), jnp.float32),
        ],
        scratch_shapes=[pltpu.VMEM((NTOK, NCL), jnp.float32)],
    )(xbf, codebooks, ln2d, centroids.reshape(NCB, 1, NCL), clc2d, bias2d)

    mesh = plsc.VectorSubcoreMesh(core_axis_name="c", subcore_axis_name="s")
    sc = functools.partial(
        pl.kernel,
        mesh=mesh,
        compiler_params=pltpu.CompilerParams(use_tc_tiling_on_sc=False, needs_layout_passes=False),
        out_type=jax.ShapeDtypeStruct((NTOK, OUT_F), jnp.float32),
        scratch_types=[
            pltpu.VMEM((TB, NCB * NCODES), jnp.float32),
            pltpu.VMEM((NCB, FQ), jnp.int32),
            pltpu.VMEM((FQ,), jnp.float32),
            pltpu.VMEM((FQ,), jnp.float32),
            pltpu.VMEM((TB, FQ), jnp.float32),
        ],
    )(_sc_body)
    y = sc(pt, addrs_q, mbias.reshape(NQ, FQ), mask.reshape(NQ, FQ))

    return y.reshape(*shape[:-1], OUT_F)


# speedup vs baseline: 4.8360x; 3.5166x over previous
import functools

import jax
import jax.numpy as jnp
from jax import lax
from jax.experimental import pallas as pl
from jax.experimental.pallas import tpu as pltpu
from jax.experimental.pallas import tpu_sc as plsc

IN_F = 2048
OUT_F = 4096
NCB = 16
NCODES = 256
DSUB = IN_F // NCB
NCL = 32
EPS = 1e-5

NTOK = 2048
RB = 256
SC_F = 512
TC_F = OUT_F - SC_F
CB = 512

NW = 32
TB = 16
TPW = NTOK // NW
NBLK = TPW // TB
GRP = SC_F // 16


def _pqa_body(x_ref, cb_ref, ln_ref, cent_ref, clcol_ref, bias_ref,
              pt_ref, mask_ref, mbias_ref, dots_ref):
    c = pl.program_id(0)
    x = x_ref[...]
    cb = cb_ref[0]
    p = jax.lax.dot_general(x, cb.astype(jnp.bfloat16), (((1,), (1,)), ((), ())),
                            preferred_element_type=jnp.float32)
    pt_ref[...] = p

    cents = cent_ref[0]
    iota = jax.lax.broadcasted_iota(jnp.int32, (NCL, NCODES), 1)
    oh = (cents[0][:, None] == iota).astype(jnp.bfloat16)
    d = jax.lax.dot_general(p.astype(jnp.bfloat16), oh, (((1,), (1,)), ((), ())),
                            preferred_element_type=jnp.float32)

    @pl.when(c == 0)
    def _():
        dots_ref[...] = jnp.zeros_like(dots_ref)
    dots_ref[...] += d

    @pl.when(c == pl.num_programs(0) - 1)
    def _fin():
        dt = dots_ref[...]
        mean = jnp.mean(dt, axis=1, keepdims=True)
        var = jnp.mean((dt - mean) ** 2, axis=1, keepdims=True)
        s = (dt - mean) * jax.lax.rsqrt(var + EPS) * ln_ref[...]
        rowmax = jnp.max(s, axis=1, keepdims=True)
        iota1 = jax.lax.broadcasted_iota(jnp.int32, (NTOK, NCL), 1)
        am = jnp.min(jnp.where(s >= rowmax, iota1, NCL), axis=1, keepdims=True)
        hit = (am == iota1).astype(jnp.float32)
        sel = jnp.max(hit, axis=0, keepdims=True)
        clc = clcol_ref[...]
        eq = (clc == jax.lax.broadcasted_iota(jnp.int32, (NCL, OUT_F), 0)
              ).astype(jnp.float32)
        m = jnp.max(eq * sel.reshape(NCL, 1), axis=0, keepdims=True)
        mask_ref[...] = m
        mbias_ref[...] = m * bias_ref[...]


def _sc_body(pt_hbm, addr_hbm, mb_hbm, mk_hbm, out_hbm,
             p_v, addr_v, mb_v, mk_v, o_v):
    wid = lax.axis_index("s") * 2 + lax.axis_index("c")
    pltpu.sync_copy(addr_hbm, addr_v)
    pltpu.sync_copy(mb_hbm, mb_v)
    pltpu.sync_copy(mk_hbm, mk_v)

    def blk(b, _):
        t0 = wid * TPW + b * TB
        pltpu.sync_copy(pt_hbm.at[pl.ds(t0, TB), :], p_v)

        @plsc.parallel_loop(0, GRP, unroll=2)
        def grp(g):
            a = [addr_v[cc, pl.ds(g * 16, 16)] for cc in range(NCB)]
            mb = mb_v[pl.ds(g * 16, 16)]
            mk = mk_v[pl.ds(g * 16, 16)]
            for t in range(TB):
                row = jnp.full((16,), t, jnp.int32)
                vs = [plsc.load_gather(p_v, [row, a[cc]]) for cc in range(NCB)]
                while len(vs) > 1:
                    nxt = [vs[i] + vs[i + 1] for i in range(0, len(vs) - 1, 2)]
                    if len(vs) % 2:
                        nxt.append(vs[-1])
                    vs = nxt
                o_v[t, pl.ds(g * 16, 16)] = vs[0] * mk + mb
        pltpu.sync_copy(o_v, out_hbm.at[pl.ds(t0, TB), :])
        return 0

    lax.fori_loop(0, NBLK, blk, 0)


def _main_body(codes_ref, x_ref, mbias_ref, mask_ref, cb_ref, out_ref, w_ref):
    r = pl.program_id(0)

    @pl.when(r == 0)
    def _build_w():
        cb = cb_ref[...]
        iota = jax.lax.broadcasted_iota(jnp.int32, (CB, NCODES), 1)
        for j in range(TC_F // CB):
            codes = codes_ref[:, j * CB:(j + 1) * CB]
            for k in range(NCB):
                oh = (codes[k][:, None] == iota).astype(jnp.bfloat16)
                w_ref[j * CB:(j + 1) * CB, k * DSUB:(k + 1) * DSUB] = jnp.dot(
                    oh, cb[k], preferred_element_type=jnp.float32).astype(jnp.bfloat16)

    x = x_ref[...]
    w = w_ref[...]
    y = jax.lax.dot_general(x, w, (((1,), (1,)), ((), ())),
                            preferred_element_type=jnp.float32)
    out_ref[...] = y * mask_ref[...] + mbias_ref[...]


def kernel(x, codebooks, bias, ln_weight, centroids, codes, indices):
    shape = x.shape
    xin = x.reshape(-1, shape[-1])
    n = xin.shape[0]
    ncl, csz = indices.shape
    xbf = xin.astype(jnp.bfloat16)
    cluster_of_col = jnp.zeros((OUT_F,), jnp.int32).at[indices.reshape(-1)].set(
        jnp.repeat(jnp.arange(ncl, dtype=jnp.int32), csz))
    clc2d = cluster_of_col.reshape(1, OUT_F)
    ln2d = ln_weight.reshape(1, NCL)
    bias2d = bias.reshape(1, OUT_F)
    addrs = (codes.astype(jnp.int32)
             + jnp.arange(NCB, dtype=jnp.int32)[:, None] * NCODES)

    pt, mask, mbias = pl.pallas_call(
        _pqa_body,
        grid=(NCB,),
        in_specs=[
            pl.BlockSpec((NTOK, DSUB), lambda c: (0, c)),
            pl.BlockSpec((1, NCODES, DSUB), lambda c: (c, 0, 0)),
            pl.BlockSpec((1, NCL), lambda c: (0, 0)),
            pl.BlockSpec((1, 1, NCL), lambda c: (c, 0, 0)),
            pl.BlockSpec((1, OUT_F), lambda c: (0, 0)),
            pl.BlockSpec((1, OUT_F), lambda c: (0, 0)),
        ],
        out_specs=[
            pl.BlockSpec((NTOK, NCODES), lambda c: (0, c)),
            pl.BlockSpec((1, OUT_F), lambda c: (0, 0)),
            pl.BlockSpec((1, OUT_F), lambda c: (0, 0)),
        ],
        out_shape=[
            jax.ShapeDtypeStruct((NTOK, NCB * NCODES), jnp.float32),
            jax.ShapeDtypeStruct((1, OUT_F), jnp.float32),
            jax.ShapeDtypeStruct((1, OUT_F), jnp.float32),
        ],
        scratch_shapes=[pltpu.VMEM((NTOK, NCL), jnp.float32)],
    )(xbf, codebooks, ln2d, centroids.reshape(NCB, 1, NCL), clc2d, bias2d)

    mesh = plsc.VectorSubcoreMesh(core_axis_name="c", subcore_axis_name="s")
    sc = functools.partial(
        pl.kernel,
        mesh=mesh,
        compiler_params=pltpu.CompilerParams(needs_layout_passes=False),
        out_type=jax.ShapeDtypeStruct((NTOK, SC_F), jnp.float32),
        scratch_types=[
            pltpu.VMEM((TB, NCB * NCODES), jnp.float32),
            pltpu.VMEM((NCB, SC_F), jnp.int32),
            pltpu.VMEM((SC_F,), jnp.float32),
            pltpu.VMEM((SC_F,), jnp.float32),
            pltpu.VMEM((TB, SC_F), jnp.float32),
        ],
    )(_sc_body)
    y_sc = sc(pt, addrs[:, TC_F:], mbias[0, TC_F:], mask[0, TC_F:])

    y_tc = pl.pallas_call(
        _main_body,
        grid=(n // RB,),
        in_specs=[
            pl.BlockSpec((NCB, TC_F), lambda r: (0, 0)),
            pl.BlockSpec((RB, IN_F), lambda r: (r, 0)),
            pl.BlockSpec((1, TC_F), lambda r: (0, 0)),
            pl.BlockSpec((1, TC_F), lambda r: (0, 0)),
            pl.BlockSpec((NCB, NCODES, DSUB), lambda r: (0, 0, 0)),
        ],
        out_specs=pl.BlockSpec((RB, TC_F), lambda r: (r, 0)),
        out_shape=jax.ShapeDtypeStruct((n, TC_F), jnp.float32),
        scratch_shapes=[pltpu.VMEM((TC_F, IN_F), jnp.bfloat16)],
    )(codes[:, :TC_F], xbf, mbias[:, :TC_F], mask[:, :TC_F], codebooks)

    y = jnp.concatenate([y_tc, y_sc], axis=1)
    return y.reshape(*shape[:-1], OUT_F)
